# baseline (speedup 1.0000x reference)
import jax
import jax.numpy as jnp
from jax.experimental import pallas as pl
from jax.experimental.pallas import tpu as pltpu

_REP = 8


def _make_body(bs, rep):
    def body(emb_ref, out_ref, stage_ref, copy_sem, out_sem):
        stages = [
            pltpu.make_async_copy(emb_ref, stage_ref.at[r], copy_sem)
            for r in range(rep)
        ]
        for c in stages:
            c.start()
        for c in stages:
            c.wait()
        n = bs // rep
        copies = [
            pltpu.make_async_copy(
                stage_ref, out_ref.at[pl.ds(g * rep, rep)], out_sem
            )
            for g in range(n)
        ]
        for c in copies:
            c.start()
        for c in copies:
            c.wait()

    return body


def kernel(mask, embed_weight):
    bs, t = mask.shape
    n_embed, d = embed_weight.shape
    rep = _REP if bs % _REP == 0 else 1

    out = pl.pallas_call(
        _make_body(bs, rep),
        in_specs=[pl.BlockSpec(memory_space=pltpu.MemorySpace.VMEM)],
        out_specs=pl.BlockSpec(memory_space=pl.ANY),
        out_shape=jax.ShapeDtypeStruct((bs, t, d), embed_weight.dtype),
        scratch_shapes=[
            pltpu.VMEM((rep, t, d), embed_weight.dtype),
            pltpu.SemaphoreType.DMA,
            pltpu.SemaphoreType.DMA,
        ],
    )(embed_weight[:t])
    return out
